# baseline (speedup 1.0000x reference)
import functools

import jax
import jax.numpy as jnp
from jax import lax
from jax.experimental import pallas as pl
from jax.experimental.pallas import tpu as pltpu
from jax.experimental.pallas import tpu_sc as plsc

_L = 16
_T = 8


def _pad_width(w):
    return (w + _T - 1) // _T * _T


def kernel(store_idx, menu_idx, holiday_idx, W_store, W_menu, W_holiday):
    B = store_idx.shape[0]
    Ds = W_store.shape[1]
    Dm = W_menu.shape[1]
    Dh = W_holiday.shape[1]
    D = Ds + Dm + Dh
    Dsp, Dmp, Dhp = _pad_width(Ds), _pad_width(Dm), _pad_width(Dh)

    info = plsc.get_sparse_core_info()
    NW = info.num_subcores
    nchunk = 2 * info.num_cores // 2
    bpw = B // (NW * nchunk)

    mesh = plsc.VectorSubcoreMesh(core_axis_name="c", subcore_axis_name="s",
                                  num_cores=1)

    @functools.partial(
        pl.kernel,
        mesh=mesh,
        out_type=jax.ShapeDtypeStruct((B, D), jnp.float32),
        compiler_params=pltpu.CompilerParams(use_tc_tiling_on_sc=False,
                                             needs_layout_passes=False),
        scratch_types=[
            pltpu.VMEM((bpw,), jnp.int32),
            pltpu.VMEM((bpw,), jnp.int32),
            pltpu.VMEM((bpw,), jnp.int32),
            pltpu.VMEM((bpw, Dsp), jnp.float32),
            pltpu.VMEM((bpw, Dmp), jnp.float32),
            pltpu.VMEM((bpw, Dhp), jnp.float32),
            pltpu.VMEM((bpw, D), jnp.float32),
            pltpu.SemaphoreType.DMA,
            pltpu.SemaphoreType.DMA,
            pltpu.SemaphoreType.DMA,
        ],
    )
    def emb_kernel(s_idx, m_idx, h_idx, ws, wm, wh, out,
                   si_v, mi_v, hi_v, sr_v, mr_v, hr_v, cat_v,
                   sem1, sem2, sem3):
        wid = lax.axis_index("s")

        def windows(width):
            w = list(range(0, max(width - _L, 0) + 1, _L))
            if w[-1] != width - _L:
                w.append(width - _L)
            return w

        def assemble(src_v, width, col0):
            offs = windows(width)

            def body(r, _):
                for c0 in offs:
                    cat_v[r, pl.ds(col0 + c0, _L)] = src_v[r, pl.ds(c0, _L)]
                return 0

            lax.fori_loop(0, bpw, body, 0)

        for chunk in range(nchunk):
            base = (wid * nchunk + chunk) * bpw
            pltpu.sync_copy(s_idx.at[pl.ds(base, bpw)], si_v)
            pltpu.sync_copy(m_idx.at[pl.ds(base, bpw)], mi_v)
            pltpu.sync_copy(h_idx.at[pl.ds(base, bpw)], hi_v)
            c1 = pltpu.async_copy(ws.at[si_v], sr_v, sem1)
            c2 = pltpu.async_copy(wm.at[mi_v], mr_v, sem2)
            c3 = pltpu.async_copy(wh.at[hi_v], hr_v, sem3)
            c1.wait()
            assemble(sr_v, Ds, 0)
            c2.wait()
            assemble(mr_v, Dm, Ds)
            c3.wait()
            assemble(hr_v, Dh, Ds + Dm)
            pltpu.sync_copy(cat_v, out.at[pl.ds(base, bpw)])

    pad = lambda w, wp: jnp.pad(w, ((0, 0), (0, wp - w.shape[1])))
    return emb_kernel(store_idx.astype(jnp.int32),
                      menu_idx.astype(jnp.int32),
                      holiday_idx.astype(jnp.int32),
                      pad(W_store, Dsp), pad(W_menu, Dmp), pad(W_holiday, Dhp))


# speedup vs baseline: 1.1284x; 1.1284x over previous
import functools

import jax
import jax.numpy as jnp
from jax import lax
from jax.experimental import pallas as pl
from jax.experimental.pallas import tpu as pltpu
from jax.experimental.pallas import tpu_sc as plsc

_L = 16
_T = 8


def _pad_width(w):
    return (w + _T - 1) // _T * _T


def kernel(store_idx, menu_idx, holiday_idx, W_store, W_menu, W_holiday):
    B = store_idx.shape[0]
    Ds = W_store.shape[1]
    Dm = W_menu.shape[1]
    Dh = W_holiday.shape[1]
    D = Ds + Dm + Dh
    Dsp, Dmp, Dhp = _pad_width(Ds), _pad_width(Dm), _pad_width(Dh)

    info = plsc.get_sparse_core_info()
    NW = info.num_cores * info.num_subcores
    bpw = B // NW

    mesh = plsc.VectorSubcoreMesh(core_axis_name="c", subcore_axis_name="s")

    @functools.partial(
        pl.kernel,
        mesh=mesh,
        out_type=jax.ShapeDtypeStruct((B, D), jnp.float32),
        compiler_params=pltpu.CompilerParams(use_tc_tiling_on_sc=False,
                                             needs_layout_passes=False),
        scratch_types=[
            pltpu.VMEM((bpw,), jnp.int32),
            pltpu.VMEM((bpw,), jnp.int32),
            pltpu.VMEM((bpw,), jnp.int32),
            pltpu.VMEM((bpw, Dsp), jnp.float32),
            pltpu.VMEM((bpw, Dmp), jnp.float32),
            pltpu.VMEM((bpw, Dhp), jnp.float32),
            pltpu.VMEM((bpw, D), jnp.float32),
            pltpu.SemaphoreType.DMA,
            pltpu.SemaphoreType.DMA,
            pltpu.SemaphoreType.DMA,
        ],
    )
    def emb_kernel(s_idx, m_idx, h_idx, ws, wm, wh, out,
                   si_v, mi_v, hi_v, sr_v, mr_v, hr_v, cat_v,
                   sem1, sem2, sem3):
        wid = lax.axis_index("s") * info.num_cores + lax.axis_index("c")
        base = wid * bpw
        pltpu.sync_copy(s_idx.at[pl.ds(base, bpw)], si_v)
        pltpu.sync_copy(m_idx.at[pl.ds(base, bpw)], mi_v)
        pltpu.sync_copy(h_idx.at[pl.ds(base, bpw)], hi_v)
        c1 = pltpu.async_copy(ws.at[si_v], sr_v, sem1)
        c2 = pltpu.async_copy(wm.at[mi_v], mr_v, sem2)
        c3 = pltpu.async_copy(wh.at[hi_v], hr_v, sem3)

        def windows(width):
            w = list(range(0, max(width - _L, 0) + 1, _L))
            if w[-1] != width - _L:
                w.append(width - _L)
            return w

        def assemble(src_v, width, col0):
            offs = windows(width)

            def body(r, _):
                for c0 in offs:
                    cat_v[r, pl.ds(col0 + c0, _L)] = src_v[r, pl.ds(c0, _L)]
                return 0

            lax.fori_loop(0, bpw, body, 0)

        c1.wait()
        assemble(sr_v, Ds, 0)
        c2.wait()
        assemble(mr_v, Dm, Ds)
        c3.wait()
        assemble(hr_v, Dh, Ds + Dm)
        pltpu.sync_copy(cat_v, out.at[pl.ds(base, bpw)])

    pad = lambda w, wp: jnp.pad(w, ((0, 0), (0, wp - w.shape[1])))
    return emb_kernel(store_idx.astype(jnp.int32),
                      menu_idx.astype(jnp.int32),
                      holiday_idx.astype(jnp.int32),
                      pad(W_store, Dsp), pad(W_menu, Dmp), pad(W_holiday, Dhp))
